# baseline (speedup 1.0000x reference)
import functools

import numpy as np

import jax
import jax.numpy as jnp
from jax.experimental import pallas as pl
from jax.experimental.pallas import tpu as pltpu
from jax.experimental.pallas import tpu_sc as plsc

HEADS = 16
DIM_HEAD = 64
DIM = 1024
K_OUT = 256
EPS = 1e-6
N = 1025
N_PAD = 1032
P = 257
P_PAD = 264
SCALE = DIM_HEAD ** -0.5
NEG = -1e30

_f32 = jnp.float32
_i32 = jnp.int32


def _iota(shape, dim):
    return jax.lax.broadcasted_iota(_i32, shape, dim)


def _qkv_body(x_ref, w_ref, o_ref):
    o_ref[...] = jax.lax.dot_general(
        x_ref[...], w_ref[...], (((1,), (0,)), ((), ())),
        preferred_element_type=_f32)


def _qkv(x_flat, w):
    rows = x_flat.shape[0]
    bm = 344
    grid = rows // bm
    return pl.pallas_call(
        _qkv_body,
        grid=(grid,),
        in_specs=[
            pl.BlockSpec((bm, DIM), lambda i: (i, 0)),
            pl.BlockSpec((DIM, 3 * DIM), lambda i: (0, 0)),
        ],
        out_specs=pl.BlockSpec((bm, 3 * DIM), lambda i: (i, 0)),
        out_shape=jax.ShapeDtypeStruct((rows, 3 * DIM), _f32),
    )(x_flat, w)


def _sample_body(q0_ref, k_ref, v_ref, o_ref):
    q0 = q0_ref[0][:1, :]
    k = k_ref[0]
    v = v_ref[0]
    hi = jax.lax.Precision.HIGHEST
    ones = jnp.ones((1, DIM_HEAD), _f32)
    vsq = v * v
    rows = []
    nrows = []
    for h in range(HEADS):
        sl = slice(h * DIM_HEAD, (h + 1) * DIM_HEAD)
        rows.append(jax.lax.dot_general(q0[:, sl], k[:, sl],
                                        (((1,), (1,)), ((), ())),
                                        preferred_element_type=_f32))
        nrows.append(jax.lax.dot_general(ones, vsq[:, sl],
                                         (((1,), (1,)), ((), ())),
                                         precision=hi,
                                         preferred_element_type=_f32))
    dots = jnp.concatenate(rows, axis=0) * SCALE
    norms = jnp.sqrt(jnp.concatenate(nrows, axis=0))
    dots = jnp.where(_iota((HEADS, N_PAD), 1) < N, dots, NEG)
    m = jnp.max(dots, axis=1, keepdims=True)
    e = jnp.exp(dots - m)
    attn0 = e / jnp.sum(e, axis=1, keepdims=True)
    c_row = jnp.sum(attn0 * norms, axis=0, keepdims=True)
    c_sel = pltpu.roll(c_row, N_PAD - 1, 1)[:, :DIM]
    total = jnp.sum(c_sel, axis=1, keepdims=True)
    o_ref[0] = jnp.log(c_sel / (total + EPS) + EPS)


def _logits(qkv):
    return pl.pallas_call(
        _sample_body,
        grid=(4,),
        in_specs=[
            pl.BlockSpec((1, 8, DIM), lambda i: (i, 0, 0)),
            pl.BlockSpec((1, N_PAD, DIM), lambda i: (i, 0, 1)),
            pl.BlockSpec((1, N_PAD, DIM), lambda i: (i, 0, 2)),
        ],
        out_specs=pl.BlockSpec((1, 1, DIM), lambda i: (i, 0, 0)),
        out_shape=jax.ShapeDtypeStruct((4, 1, DIM), _f32),
    )(qkv, qkv, qkv)


P_SC = P_PAD + 8
_N_CHUNK = DIM // 16
_RPW = 64


def _sc_sample_build():
    mesh = plsc.VectorSubcoreMesh(core_axis_name="c", subcore_axis_name="s",
                                  num_cores=1)

    @functools.partial(
        pl.kernel, mesh=mesh,
        compiler_params=pltpu.CompilerParams(needs_layout_passes=False),
        out_type=(jax.ShapeDtypeStruct((4, P_SC), _i32),
                  jax.ShapeDtypeStruct((4, K_OUT), _i32)),
        scratch_types=[
            pltpu.VMEM((DIM,), _f32),
            pltpu.VMEM((DIM,), _f32),
            pltpu.VMEM((_RPW,), _i32),
            pltpu.VMEM((K_OUT,), _i32),
            pltpu.VMEM((DIM,), _i32),
            pltpu.VMEM((DIM,), _i32),
            pltpu.VMEM((P_SC,), _i32),
        ],
    )
    def k(lg_hbm, g_hbm, ids_hbm, samp_hbm,
          lg_v, g_v, sampw_v, samp2_v, tbl_v, rk_v, ids_v):
        w = jax.lax.axis_index("s")
        b = w // 4
        s0 = (w % 4) * _RPW
        pltpu.sync_copy(lg_hbm.at[b], lg_v)

        lane = jax.lax.iota(_i32, 16)

        def row_body(r, carry):
            pltpu.sync_copy(g_hbm.at[b, s0 + r], g_v)
            bv = jnp.full((16,), -jnp.inf, _f32)
            bi = jnp.zeros((16,), _i32)
            for c in range(_N_CHUNK):
                sc = g_v[pl.ds(c * 16, 16)] + lg_v[pl.ds(c * 16, 16)]
                upd = sc > bv
                bv = jnp.where(upd, sc, bv)
                bi = jnp.where(upd, c * 16 + lane, bi)
            _, bi_sorted = plsc.sort_key_val(bv, bi, descending=True)
            plsc.store_scatter(sampw_v, [jnp.full((16,), r, _i32)],
                               bi_sorted, mask=lane == 0)
            return carry

        jax.lax.fori_loop(0, _RPW, row_body, 0)
        pltpu.sync_copy(sampw_v, samp_hbm.at[b, pl.ds(s0, _RPW)])
        plsc.subcore_barrier()

        @pl.when(w < 4)
        def _():
            pltpu.sync_copy(samp_hbm.at[w], samp2_v)
            for c in range(_N_CHUNK):
                tbl_v[pl.ds(c * 16, 16)] = jnp.zeros((16,), _i32)
            for c in range(K_OUT // 16):
                idx16 = samp2_v[pl.ds(c * 16, 16)]
                plsc.store_scatter(tbl_v, [idx16], jnp.ones((16,), _i32))
            u = jnp.zeros((16,), _i32)
            for c in range(_N_CHUNK):
                cs = plsc.cumsum(tbl_v[pl.ds(c * 16, 16)])
                rk_v[pl.ds(c * 16, 16)] = cs + u
                u = u + plsc.cummax(jax.lax.rev(cs, (0,)))
            for c in range(P_SC // 16):
                ids_v[pl.ds(c * 16, 16)] = jnp.zeros((16,), _i32)
            for c in range(_N_CHUNK):
                pres = tbl_v[pl.ds(c * 16, 16)] > 0
                pos = (K_OUT - u) + rk_v[pl.ds(c * 16, 16)]
                val = c * 16 + lane + 1
                plsc.store_scatter(ids_v, [pos], val, mask=pres)
            pltpu.sync_copy(ids_v, ids_hbm.at[w])

    return k


_SC_SAMPLE = _sc_sample_build()


def _attn_out_body(ids_ref, q_ref, k_ref, v_ref, wo_ref, bo_ref, o_ref):
    ids = ids_ref[0]
    q = q_ref[0]
    k = k_ref[0]
    v = v_ref[0]
    GT = (_iota((N_PAD, P_PAD), 0) == ids).astype(_f32)
    qs = jax.lax.dot_general(GT, q, (((0,), (0,)), ((), ())),
                             preferred_element_type=_f32)
    col = _iota((P_PAD, N_PAD), 1)
    outs = []
    for h in range(HEADS):
        sl = slice(h * DIM_HEAD, (h + 1) * DIM_HEAD)
        dh = jax.lax.dot_general(qs[:, sl], k[:, sl], (((1,), (1,)), ((), ())),
                                 preferred_element_type=_f32) * SCALE
        dh = jnp.where(col < N, dh, NEG)
        mh = jnp.max(dh, axis=1, keepdims=True)
        eh = jnp.exp(dh - mh)
        ah = eh / jnp.sum(eh, axis=1, keepdims=True)
        outs.append(jax.lax.dot_general(ah, v[:, sl], (((1,), (0,)), ((), ())),
                                        preferred_element_type=_f32))
    av = jnp.concatenate(outs, axis=1)
    o_ref[0] = jax.lax.dot_general(av, wo_ref[...], (((1,), (0,)), ((), ())),
                                   preferred_element_type=_f32) + bo_ref[...]


def _attn_out(ids, qkv, w_out, b_out):
    return pl.pallas_call(
        _attn_out_body,
        grid=(4,),
        in_specs=[
            pl.BlockSpec((1, 1, P_PAD), lambda i: (i, 0, 0)),
            pl.BlockSpec((1, N_PAD, DIM), lambda i: (i, 0, 0)),
            pl.BlockSpec((1, N_PAD, DIM), lambda i: (i, 0, 1)),
            pl.BlockSpec((1, N_PAD, DIM), lambda i: (i, 0, 2)),
            pl.BlockSpec((DIM, DIM), lambda i: (0, 0)),
            pl.BlockSpec((1, DIM), lambda i: (0, 0)),
        ],
        out_specs=pl.BlockSpec((1, P_PAD, DIM), lambda i: (i, 0, 0)),
        out_shape=jax.ShapeDtypeStruct((4, P_PAD, DIM), _f32),
    )(ids, qkv, qkv, qkv, w_out, b_out)


def kernel(x, mask, W_qkv, W_out, b_out):
    b, n, _ = x.shape
    xp = jnp.pad(x, ((0, 0), (0, N_PAD - n), (0, 0)))
    qkv = _qkv(xp.reshape(b * N_PAD, DIM), W_qkv).reshape(b, N_PAD, 3 * DIM)

    u = jax.random.uniform(jax.random.key(42), (b, K_OUT, DIM), dtype=_f32)
    gumbel = -jnp.log(-jnp.log(u + EPS) + EPS)

    logits = _logits(qkv).reshape(b, DIM)
    ids_sc, _ = _SC_SAMPLE(logits, gumbel)
    ids_pad = ids_sc[:, :P_PAD].reshape(b, 1, P_PAD)
    out_pad = _attn_out(ids_pad, qkv, W_out, b_out.reshape(1, DIM))

    sampled_ids = ids_pad.reshape(b, P_PAD)[:, :P]
    out = out_pad[:, :P, :]
    new_mask = jnp.concatenate(
        [jnp.ones((b, 1), bool), sampled_ids[:, 1:] != 0], axis=1)
    return out, new_mask, sampled_ids


# speedup vs baseline: 1.1481x; 1.1481x over previous
import functools

import numpy as np

import jax
import jax.numpy as jnp
from jax.experimental import pallas as pl
from jax.experimental.pallas import tpu as pltpu
from jax.experimental.pallas import tpu_sc as plsc

HEADS = 16
DIM_HEAD = 64
DIM = 1024
K_OUT = 256
EPS = 1e-6
N = 1025
N_PAD = 1032
P = 257
P_PAD = 264
SCALE = DIM_HEAD ** -0.5
NEG = -1e30

_f32 = jnp.float32
_i32 = jnp.int32


def _iota(shape, dim):
    return jax.lax.broadcasted_iota(_i32, shape, dim)


def _qkv_body(x_ref, w_ref, o_ref):
    o_ref[...] = jax.lax.dot_general(
        x_ref[...], w_ref[...], (((1,), (0,)), ((), ())),
        preferred_element_type=_f32)


def _qkv(x_flat, w):
    rows = x_flat.shape[0]
    bm = 344
    grid = rows // bm
    return pl.pallas_call(
        _qkv_body,
        grid=(grid,),
        in_specs=[
            pl.BlockSpec((bm, DIM), lambda i: (i, 0)),
            pl.BlockSpec((DIM, 3 * DIM), lambda i: (0, 0)),
        ],
        out_specs=pl.BlockSpec((bm, 3 * DIM), lambda i: (i, 0)),
        out_shape=jax.ShapeDtypeStruct((rows, 3 * DIM), _f32),
    )(x_flat, w)


def _sample_body(q0_ref, k_ref, v_ref, o_ref):
    q0 = q0_ref[0][:1, :]
    k = k_ref[0]
    v = v_ref[0]
    hi = jax.lax.Precision.HIGHEST
    ones = jnp.ones((1, DIM_HEAD), _f32)
    vsq = v * v
    rows = []
    nrows = []
    for h in range(HEADS):
        sl = slice(h * DIM_HEAD, (h + 1) * DIM_HEAD)
        rows.append(jax.lax.dot_general(q0[:, sl], k[:, sl],
                                        (((1,), (1,)), ((), ())),
                                        preferred_element_type=_f32))
        nrows.append(jax.lax.dot_general(ones, vsq[:, sl],
                                         (((1,), (1,)), ((), ())),
                                         precision=hi,
                                         preferred_element_type=_f32))
    dots = jnp.concatenate(rows, axis=0) * SCALE
    norms = jnp.sqrt(jnp.concatenate(nrows, axis=0))
    dots = jnp.where(_iota((HEADS, N_PAD), 1) < N, dots, NEG)
    m = jnp.max(dots, axis=1, keepdims=True)
    e = jnp.exp(dots - m)
    attn0 = e / jnp.sum(e, axis=1, keepdims=True)
    c_row = jnp.sum(attn0 * norms, axis=0, keepdims=True)
    c_sel = pltpu.roll(c_row, N_PAD - 1, 1)[:, :DIM]
    total = jnp.sum(c_sel, axis=1, keepdims=True)
    o_ref[0] = jnp.log(c_sel / (total + EPS) + EPS)


def _logits(qkv):
    return pl.pallas_call(
        _sample_body,
        grid=(4,),
        in_specs=[
            pl.BlockSpec((1, 8, DIM), lambda i: (i, 0, 0)),
            pl.BlockSpec((1, N_PAD, DIM), lambda i: (i, 0, 1)),
            pl.BlockSpec((1, N_PAD, DIM), lambda i: (i, 0, 2)),
        ],
        out_specs=pl.BlockSpec((1, 1, DIM), lambda i: (i, 0, 0)),
        out_shape=jax.ShapeDtypeStruct((4, 1, DIM), _f32),
    )(qkv, qkv, qkv)


P_SC = P_PAD + 8
_N_CHUNK = DIM // 16
_RPW = 64


def _sc_sample_build():
    mesh = plsc.VectorSubcoreMesh(core_axis_name="c", subcore_axis_name="s",
                                  num_cores=1)

    @functools.partial(
        pl.kernel, mesh=mesh,
        compiler_params=pltpu.CompilerParams(needs_layout_passes=False),
        out_type=(jax.ShapeDtypeStruct((4, P_SC), _i32),
                  jax.ShapeDtypeStruct((4, K_OUT), _i32)),
        scratch_types=[
            pltpu.VMEM((DIM,), _f32),
            pltpu.VMEM((_RPW, DIM), _f32),
            pltpu.VMEM((_RPW,), _i32),
            pltpu.VMEM((K_OUT,), _i32),
            pltpu.VMEM((DIM,), _i32),
            pltpu.VMEM((DIM,), _i32),
            pltpu.VMEM((P_SC,), _i32),
        ],
    )
    def k(lg_hbm, g_hbm, ids_hbm, samp_hbm,
          lg_v, g_v, sampw_v, samp2_v, tbl_v, rk_v, ids_v):
        w = jax.lax.axis_index("s")
        b = w // 4
        s0 = (w % 4) * _RPW
        pltpu.sync_copy(lg_hbm.at[b], lg_v)

        lane = jax.lax.iota(_i32, 16)
        pltpu.sync_copy(g_hbm.at[b, pl.ds(s0, _RPW)], g_v)

        def row_body(r, carry):
            bv = jnp.full((16,), -jnp.inf, _f32)
            bi = jnp.zeros((16,), _i32)
            for c in range(_N_CHUNK):
                sc = g_v[r, pl.ds(c * 16, 16)] + lg_v[pl.ds(c * 16, 16)]
                upd = sc > bv
                bv = jnp.where(upd, sc, bv)
                bi = jnp.where(upd, c * 16 + lane, bi)
            _, bi_sorted = plsc.sort_key_val(bv, bi, descending=True)
            plsc.store_scatter(sampw_v, [jnp.full((16,), r, _i32)],
                               bi_sorted, mask=lane == 0)
            return carry

        jax.lax.fori_loop(0, _RPW, row_body, 0)
        pltpu.sync_copy(sampw_v, samp_hbm.at[b, pl.ds(s0, _RPW)])
        plsc.subcore_barrier()

        @pl.when(w < 4)
        def _():
            pltpu.sync_copy(samp_hbm.at[w], samp2_v)
            for c in range(_N_CHUNK):
                tbl_v[pl.ds(c * 16, 16)] = jnp.zeros((16,), _i32)
            for c in range(K_OUT // 16):
                idx16 = samp2_v[pl.ds(c * 16, 16)]
                plsc.store_scatter(tbl_v, [idx16], jnp.ones((16,), _i32))
            u = jnp.zeros((16,), _i32)
            for c in range(_N_CHUNK):
                cs = plsc.cumsum(tbl_v[pl.ds(c * 16, 16)])
                rk_v[pl.ds(c * 16, 16)] = cs + u
                u = u + plsc.cummax(jax.lax.rev(cs, (0,)))
            for c in range(P_SC // 16):
                ids_v[pl.ds(c * 16, 16)] = jnp.zeros((16,), _i32)
            for c in range(_N_CHUNK):
                pres = tbl_v[pl.ds(c * 16, 16)] > 0
                pos = (K_OUT - u) + rk_v[pl.ds(c * 16, 16)]
                val = c * 16 + lane + 1
                plsc.store_scatter(ids_v, [pos], val, mask=pres)
            pltpu.sync_copy(ids_v, ids_hbm.at[w])

    return k


_SC_SAMPLE = _sc_sample_build()


def _attn_out_body(ids_ref, q_ref, k_ref, v_ref, wo_ref, bo_ref, o_ref):
    ids = ids_ref[0]
    q = q_ref[0]
    k = k_ref[0]
    v = v_ref[0]
    GT = (_iota((N_PAD, P_PAD), 0) == ids).astype(_f32)
    qs = jax.lax.dot_general(GT, q, (((0,), (0,)), ((), ())),
                             preferred_element_type=_f32)
    col = _iota((P_PAD, N_PAD), 1)
    outs = []
    for h in range(HEADS):
        sl = slice(h * DIM_HEAD, (h + 1) * DIM_HEAD)
        dh = jax.lax.dot_general(qs[:, sl], k[:, sl], (((1,), (1,)), ((), ())),
                                 preferred_element_type=_f32) * SCALE
        dh = jnp.where(col < N, dh, NEG)
        mh = jnp.max(dh, axis=1, keepdims=True)
        eh = jnp.exp(dh - mh)
        ah = eh / jnp.sum(eh, axis=1, keepdims=True)
        outs.append(jax.lax.dot_general(ah, v[:, sl], (((1,), (0,)), ((), ())),
                                        preferred_element_type=_f32))
    av = jnp.concatenate(outs, axis=1)
    o_ref[0] = jax.lax.dot_general(av, wo_ref[...], (((1,), (0,)), ((), ())),
                                   preferred_element_type=_f32) + bo_ref[...]


def _attn_out(ids, qkv, w_out, b_out):
    return pl.pallas_call(
        _attn_out_body,
        grid=(4,),
        in_specs=[
            pl.BlockSpec((1, 1, P_PAD), lambda i: (i, 0, 0)),
            pl.BlockSpec((1, N_PAD, DIM), lambda i: (i, 0, 0)),
            pl.BlockSpec((1, N_PAD, DIM), lambda i: (i, 0, 1)),
            pl.BlockSpec((1, N_PAD, DIM), lambda i: (i, 0, 2)),
            pl.BlockSpec((DIM, DIM), lambda i: (0, 0)),
            pl.BlockSpec((1, DIM), lambda i: (0, 0)),
        ],
        out_specs=pl.BlockSpec((1, P_PAD, DIM), lambda i: (i, 0, 0)),
        out_shape=jax.ShapeDtypeStruct((4, P_PAD, DIM), _f32),
    )(ids, qkv, qkv, qkv, w_out, b_out)


def kernel(x, mask, W_qkv, W_out, b_out):
    b, n, _ = x.shape
    xp = jnp.pad(x, ((0, 0), (0, N_PAD - n), (0, 0)))
    qkv = _qkv(xp.reshape(b * N_PAD, DIM), W_qkv).reshape(b, N_PAD, 3 * DIM)

    u = jax.random.uniform(jax.random.key(42), (b, K_OUT, DIM), dtype=_f32)
    gumbel = -jnp.log(-jnp.log(u + EPS) + EPS)

    logits = _logits(qkv).reshape(b, DIM)
    ids_sc, _ = _SC_SAMPLE(logits, gumbel)
    ids_pad = ids_sc[:, :P_PAD].reshape(b, 1, P_PAD)
    out_pad = _attn_out(ids_pad, qkv, W_out, b_out.reshape(1, DIM))

    sampled_ids = ids_pad.reshape(b, P_PAD)[:, :P]
    out = out_pad[:, :P, :]
    new_mask = jnp.concatenate(
        [jnp.ones((b, 1), bool), sampled_ids[:, 1:] != 0], axis=1)
    return out, new_mask, sampled_ids
